# baseline (speedup 1.0000x reference)
import functools

import jax
import jax.numpy as jnp
from jax import lax
from jax.experimental import pallas as pl
from jax.experimental.pallas import tpu as pltpu
from jax.experimental.pallas import tpu_sc as plsc

VOCAB = 1000
D = 64
B = 16384
C = 10
L = 16
NC, NS = 2, 16
NW = NC * NS
RW = B // NW
NB = 32
NI = RW // NB
EV = 64

_LN2 = 0.6931471805599453


def _log_f32(x):
    bits = lax.bitcast_convert_type(x, jnp.int32)
    e = (bits >> 23) - 127
    mant = lax.bitcast_convert_type(
        (bits & jnp.int32(0x7FFFFF)) | jnp.int32(0x3F800000), jnp.float32)
    t = (mant - 1.0) / (mant + 1.0)
    t2 = t * t
    p = 1.0 + t2 * (0.3333333333 + t2 * (0.2 + t2 * 0.14285714285))
    return e.astype(jnp.float32) * _LN2 + 2.0 * t * p


def _sc_body(table_hbm, idx_hbm, out_hbm,
             table_v, idx_v, obufs, m_loc, s_loc, m_v, s_v, lse_v,
             m_sh, s_sh, sem_o):
    cid = lax.axis_index("c")
    sid = lax.axis_index("s")
    wid = sid * NC + cid
    row0 = wid * RW

    pltpu.sync_copy(idx_hbm.at[pl.ds(wid * (RW * C // 128), RW * C // 128)],
                    idx_v)
    pltpu.sync_copy(table_hbm, table_v)

    vstart = jnp.where(sid == NS - 1, VOCAB - EV, sid * EV)
    lanes = lax.broadcasted_iota(jnp.int32, (L,), 0)

    @plsc.parallel_loop(0, EV // L, unroll=2)
    def stat_body(grp):
        rr = vstart + lanes + grp * L
        rows16 = rr >> 1
        cols16 = (rr & 1) << 6
        macc = plsc.load_gather(table_v, [rows16, cols16])
        for d in range(1, D):
            macc = jnp.maximum(
                macc, plsc.load_gather(table_v, [rows16, cols16 + d]))
        sacc = jnp.zeros((L,), jnp.float32)
        for d in range(D):
            v = plsc.load_gather(table_v, [rows16, cols16 + d])
            sacc = sacc + jnp.exp(v - macc)
        m_loc[pl.ds(grp * L, L)] = macc
        s_loc[pl.ds(grp * L, L)] = sacc

    pltpu.sync_copy(m_loc, m_sh.at[pl.ds(vstart, EV)])
    pltpu.sync_copy(s_loc, s_sh.at[pl.ds(vstart, EV)])
    plsc.subcore_barrier()
    pltpu.sync_copy(m_sh, m_v)
    pltpu.sync_copy(s_sh, s_v)

    @plsc.parallel_loop(0, RW // L, unroll=2)
    def lse_body(grp):
        base16 = (lanes + grp * L) * C
        idxs = []
        for c in range(C):
            q = base16 + c
            idxs.append(plsc.load_gather(idx_v, [q >> 7, q & 127]))
        ms = [plsc.load_gather(m_v, [ix]) for ix in idxs]
        m16 = ms[0]
        for c in range(1, C):
            m16 = jnp.maximum(m16, ms[c])
        z16 = jnp.zeros((L,), jnp.float32)
        for c in range(C):
            s = plsc.load_gather(s_v, [idxs[c]])
            z16 = z16 + s * jnp.exp(ms[c] - m16)
        lse_v[pl.ds(grp * L, L)] = 0.0 - (m16 + _log_f32(z16))

    def drain(p):
        pltpu.make_async_copy(
            obufs[p], out_hbm.at[pl.ds(row0, NB)], sem_o[p]).wait()

    def do_chunk(it, p):
        @plsc.parallel_loop(0, NB, unroll=4)
        def row_body(r):
            nls = plsc.load_gather(
                lse_v, [jnp.broadcast_to(it * NB + r, (L,)).astype(jnp.int32)])
            q = jnp.broadcast_to((it * NB + r) * C, (L,)).astype(jnp.int32) + lanes
            vv = plsc.load_gather(idx_v, [q >> 7, q & 127])
            for c in range(C):
                v = vv[c]
                u = v >> 1
                col = (v & 1) << 6
                for k in range(D // L):
                    x = table_v[u, pl.ds(col + k * L, L)]
                    obufs[p][r, pl.ds(c * D + k * L, L)] = x + nls

        pltpu.async_copy(
            obufs[p], out_hbm.at[pl.ds(row0 + it * NB, NB)], sem_o[p])

    do_chunk(0, 0)
    do_chunk(1, 1)

    def chunk_pair(g, _):
        for p in range(2):
            drain(p)
            do_chunk(g * 2 + p, p)
        return _

    lax.fori_loop(1, NI // 2, chunk_pair, 0)
    drain(0)
    drain(1)


@functools.partial(
    pl.kernel,
    out_type=jax.ShapeDtypeStruct((B, C * D), jnp.float32),
    mesh=plsc.VectorSubcoreMesh(core_axis_name="c", subcore_axis_name="s"),
    compiler_params=pltpu.CompilerParams(
        needs_layout_passes=False, use_tc_tiling_on_sc=True),
    scratch_types=[
        pltpu.VMEM((VOCAB // 2, 2 * D), jnp.float32),
        pltpu.VMEM((RW * C // 128, 128), jnp.int32),
        pltpu.VMEM((NB, C * D), jnp.float32),
        pltpu.VMEM((NB, C * D), jnp.float32),
        pltpu.VMEM((EV,), jnp.float32),
        pltpu.VMEM((EV,), jnp.float32),
        pltpu.VMEM((1024,), jnp.float32),
        pltpu.VMEM((1024,), jnp.float32),
        pltpu.VMEM((RW,), jnp.float32),
        pltpu.VMEM_SHARED((1024,), jnp.float32),
        pltpu.VMEM_SHARED((1024,), jnp.float32),
        pltpu.SemaphoreType.DMA,
        pltpu.SemaphoreType.DMA,
    ],
)
def _sc_kernel(table_hbm, idx_hbm, out_hbm,
               table_v, idx_v, obuf_a, obuf_b, m_loc, s_loc, m_v, s_v,
               lse_v, m_sh, s_sh, sem_oa, sem_ob):
    _sc_body(table_hbm, idx_hbm, out_hbm,
             table_v, idx_v, [obuf_a, obuf_b], m_loc, s_loc, m_v, s_v,
             lse_v, m_sh, s_sh, [sem_oa, sem_ob])


def kernel(batch_of_context_vectors, embedding_weight):
    return _sc_kernel(embedding_weight.reshape(VOCAB // 2, 2 * D),
                      batch_of_context_vectors.reshape(B * C // 128, 128))


# speedup vs baseline: 1.0546x; 1.0546x over previous
import functools

import jax
import jax.numpy as jnp
from jax import lax
from jax.experimental import pallas as pl
from jax.experimental.pallas import tpu as pltpu
from jax.experimental.pallas import tpu_sc as plsc

VOCAB = 1000
D = 64
B = 16384
C = 10
L = 16
NC, NS = 2, 16
NW = NC * NS
RW = B // NW
NB = 32
NI = RW // NB
EV = 64

_LN2 = 0.6931471805599453


def _log_f32(x):
    bits = lax.bitcast_convert_type(x, jnp.int32)
    e = (bits >> 23) - 127
    mant = lax.bitcast_convert_type(
        (bits & jnp.int32(0x7FFFFF)) | jnp.int32(0x3F800000), jnp.float32)
    t = (mant - 1.0) / (mant + 1.0)
    t2 = t * t
    p = 1.0 + t2 * (0.3333333333 + t2 * (0.2 + t2 * 0.14285714285))
    return e.astype(jnp.float32) * _LN2 + 2.0 * t * p


def _sc_body(table_hbm, idx_hbm, out_hbm,
             table_v, idx_v, obufs, m_loc, s_loc, m_v, s_v, lse_v,
             m_sh, s_sh, sem_o):
    cid = lax.axis_index("c")
    sid = lax.axis_index("s")
    wid = sid * NC + cid
    row0 = wid * RW

    pltpu.sync_copy(idx_hbm.at[pl.ds(wid * (RW * C // 128), RW * C // 128)],
                    idx_v)
    pltpu.sync_copy(table_hbm, table_v)

    vstart = jnp.where(sid == NS - 1, VOCAB - EV, sid * EV)
    lanes = lax.broadcasted_iota(jnp.int32, (L,), 0)

    @plsc.parallel_loop(0, EV // L, unroll=2)
    def stat_body(grp):
        rr = vstart + lanes + grp * L
        rows16 = rr >> 1
        cols16 = (rr & 1) << 6
        macc = plsc.load_gather(table_v, [rows16, cols16])
        for d in range(1, D):
            macc = jnp.maximum(
                macc, plsc.load_gather(table_v, [rows16, cols16 + d]))
        sacc = jnp.zeros((L,), jnp.float32)
        for d in range(D):
            v = plsc.load_gather(table_v, [rows16, cols16 + d])
            sacc = sacc + jnp.exp(v - macc)
        m_loc[pl.ds(grp * L, L)] = macc
        s_loc[pl.ds(grp * L, L)] = sacc

    pltpu.sync_copy(m_loc, m_sh.at[pl.ds(vstart, EV)])
    pltpu.sync_copy(s_loc, s_sh.at[pl.ds(vstart, EV)])
    plsc.subcore_barrier()
    pltpu.sync_copy(m_sh, m_v)
    pltpu.sync_copy(s_sh, s_v)

    @plsc.parallel_loop(0, RW // L, unroll=2)
    def lse_body(grp):
        base16 = (lanes + grp * L) * C
        idxs = []
        for c in range(C):
            q = base16 + c
            idxs.append(plsc.load_gather(idx_v, [q >> 7, q & 127]))
        ms = [plsc.load_gather(m_v, [ix]) for ix in idxs]
        m16 = ms[0]
        for c in range(1, C):
            m16 = jnp.maximum(m16, ms[c])
        z16 = jnp.zeros((L,), jnp.float32)
        for c in range(C):
            s = plsc.load_gather(s_v, [idxs[c]])
            z16 = z16 + s * jnp.exp(ms[c] - m16)
        lse_v[pl.ds(grp * L, L)] = 0.0 - (m16 + _log_f32(z16))

    def drain(p):
        pltpu.make_async_copy(
            obufs[p], out_hbm.at[pl.ds(row0, NB)], sem_o[p]).wait()

    def do_chunk(it, p):
        @plsc.parallel_loop(0, NB, unroll=2)
        def row_body(r):
            nls = plsc.load_gather(
                lse_v, [jnp.broadcast_to(it * NB + r, (L,)).astype(jnp.int32)])
            q = jnp.broadcast_to((it * NB + r) * C, (L,)).astype(jnp.int32) + lanes
            vv = plsc.load_gather(idx_v, [q >> 7, q & 127])
            for c in range(C):
                v = vv[c]
                u = v >> 1
                col = (v & 1) << 6
                for k in range(D // L):
                    x = table_v[u, pl.ds(col + k * L, L)]
                    obufs[p][r, pl.ds(c * D + k * L, L)] = x + nls

        pltpu.async_copy(
            obufs[p], out_hbm.at[pl.ds(row0 + it * NB, NB)], sem_o[p])

    do_chunk(0, 0)
    do_chunk(1, 1)

    def chunk_pair(g, _):
        for p in range(2):
            drain(p)
            do_chunk(g * 2 + p, p)
        return _

    lax.fori_loop(1, NI // 2, chunk_pair, 0)
    drain(0)
    drain(1)


@functools.partial(
    pl.kernel,
    out_type=jax.ShapeDtypeStruct((B, C * D), jnp.float32),
    mesh=plsc.VectorSubcoreMesh(core_axis_name="c", subcore_axis_name="s"),
    compiler_params=pltpu.CompilerParams(
        needs_layout_passes=False, use_tc_tiling_on_sc=True),
    scratch_types=[
        pltpu.VMEM((VOCAB // 2, 2 * D), jnp.float32),
        pltpu.VMEM((RW * C // 128, 128), jnp.int32),
        pltpu.VMEM((NB, C * D), jnp.float32),
        pltpu.VMEM((NB, C * D), jnp.float32),
        pltpu.VMEM((EV,), jnp.float32),
        pltpu.VMEM((EV,), jnp.float32),
        pltpu.VMEM((1024,), jnp.float32),
        pltpu.VMEM((1024,), jnp.float32),
        pltpu.VMEM((RW,), jnp.float32),
        pltpu.VMEM_SHARED((1024,), jnp.float32),
        pltpu.VMEM_SHARED((1024,), jnp.float32),
        pltpu.SemaphoreType.DMA,
        pltpu.SemaphoreType.DMA,
    ],
)
def _sc_kernel(table_hbm, idx_hbm, out_hbm,
               table_v, idx_v, obuf_a, obuf_b, m_loc, s_loc, m_v, s_v,
               lse_v, m_sh, s_sh, sem_oa, sem_ob):
    _sc_body(table_hbm, idx_hbm, out_hbm,
             table_v, idx_v, [obuf_a, obuf_b], m_loc, s_loc, m_v, s_v,
             lse_v, m_sh, s_sh, [sem_oa, sem_ob])


def kernel(batch_of_context_vectors, embedding_weight):
    return _sc_kernel(embedding_weight.reshape(VOCAB // 2, 2 * D),
                      batch_of_context_vectors.reshape(B * C // 128, 128))
